# baseline (speedup 1.0000x reference)
import functools

import jax
import jax.numpy as jnp
from jax import lax
from jax.experimental import pallas as pl
from jax.experimental.pallas import tpu as pltpu
from jax.experimental.pallas import tpu_sc as plsc

N = 10000
E = 320000
IND = 128
H0 = 512
D = 128
NCLS = 16
EPS = 1e-5

NC = 2
NS = 16
NW = NC * NS
K = 128
RPW = 80
ROWS2D = NW * RPW
EPAD = ROWS2D * K
NPADROWS = 240
NPAD = N + NPADROWS
RPT = NPAD // NS
NBUF = 5
LOOKAHEAD = 3
CPT = ROWS2D // NS
DH = D // 2

_mesh = plsc.VectorSubcoreMesh(
    core_axis_name="c", subcore_axis_name="s", num_cores=NC, num_subcores=NS
)


@functools.partial(
    pl.kernel,
    out_type=jax.ShapeDtypeStruct((NC * NPAD,), jnp.float32),
    mesh=_mesh,
    scratch_types=[
        pltpu.VMEM((RPW, K), jnp.int32),
        pltpu.VMEM((K,), jnp.float32),
        pltpu.VMEM((640,), jnp.float32),
        pltpu.VMEM_SHARED((NPAD,), jnp.float32),
    ],
)
def _deg_kernel(dst_hbm, out_hbm, dstv, ones_v, buf_v, acc):
    cid = lax.axis_index("c")
    sid = lax.axis_index("s")
    wid = sid * NC + cid
    for i in range(K // 16):
        ones_v[pl.ds(i * 16, 16)] = jnp.ones((16,), jnp.float32)
    for i in range(640 // 16):
        buf_v[pl.ds(i * 16, 16)] = jnp.zeros((16,), jnp.float32)

    pltpu.sync_copy(buf_v.at[pl.ds(0, RPT)], acc.at[pl.ds(sid * RPT, RPT)])

    pltpu.sync_copy(dst_hbm.at[pl.ds(wid * RPW, RPW)], dstv)
    plsc.subcore_barrier()

    def body(j, carry):
        pltpu.sync_copy(ones_v, acc.at[dstv.at[j]], add=True)
        return carry

    lax.fori_loop(0, RPW, body, 0)
    plsc.subcore_barrier()

    pltpu.sync_copy(acc.at[pl.ds(sid * RPT, RPT)], buf_v.at[pl.ds(0, RPT)])
    pltpu.sync_copy(
        buf_v.at[pl.ds(0, RPT)], out_hbm.at[pl.ds(cid * NPAD + sid * RPT, RPT)]
    )


def _conv_body(staged, u_hbm, src_hbm, dst_hbm, out_hbm, srcv, dstv, rows, zbuf,
               acc, stage, semg, sems, semz):
    cid = lax.axis_index("c")
    sid = lax.axis_index("s")
    base = sid * RPT

    if staged:
        pltpu.sync_copy(src_hbm.at[pl.ds(sid * CPT, CPT)], srcv)
    else:
        pltpu.sync_copy(src_hbm.at[pl.ds(cid * ROWS2D + sid * CPT, CPT)], srcv)
    pltpu.sync_copy(dst_hbm.at[pl.ds(sid * CPT, CPT)], dstv)
    if staged:
        sr0 = sid * 632

        @pl.when(sid < 15)
        def _():
            pltpu.sync_copy(
                u_hbm.at[pl.ds(cid * N + sr0, 632)], stage.at[pl.ds(sr0, 632)]
            )

        @pl.when(sid == 15)
        def _():
            pltpu.sync_copy(
                u_hbm.at[pl.ds(cid * N + 15 * 632, 520)],
                stage.at[pl.ds(15 * 632, 520)],
            )
    else:
        stage = u_hbm
        for b in range(LOOKAHEAD):
            pltpu.async_copy(stage.at[srcv.at[b]], rows.at[b], semg.at[b])

    for r in range(16):
        for c in range(DH // 16):
            zbuf[r, pl.ds(c * 16, 16)] = jnp.zeros((16,), jnp.float32)

    def zero_start(i, carry):
        pltpu.async_copy(zbuf, acc.at[pl.ds(base + i * 16, 16)], semz)
        return carry

    lax.fori_loop(0, RPT // 16, zero_start, 0)

    def zero_drain(i, carry):
        pltpu.make_async_copy(zbuf, acc.at[pl.ds(base, 16)], semz).wait()
        return carry

    lax.fori_loop(0, RPT // 16, zero_drain, 0)
    plsc.subcore_barrier()
    if staged:
        for b in range(LOOKAHEAD):
            pltpu.async_copy(stage.at[srcv.at[b]], rows.at[b], semg.at[b])

    def step(g, carry):
        for b in range(NBUF):
            j = g * NBUF + b
            pltpu.make_async_copy(stage.at[srcv.at[j]], rows.at[b], semg.at[b]).wait()
            pltpu.async_copy(rows.at[b], acc.at[dstv.at[j]], sems.at[b], add=True)
            jj = j + LOOKAHEAD
            bb = (b + LOOKAHEAD) % NBUF

            @pl.when(jj < CPT)
            def _():
                @pl.when(j >= NBUF - LOOKAHEAD)
                def _():
                    pltpu.make_async_copy(
                        rows.at[bb], acc.at[dstv.at[j]], sems.at[bb]
                    ).wait()

                pltpu.async_copy(stage.at[srcv.at[jj]], rows.at[bb], semg.at[bb])

        return carry

    lax.fori_loop(0, CPT // NBUF, step, 0)
    for b in range(NBUF):
        pltpu.make_async_copy(rows.at[b], acc.at[dstv.at[0]], sems.at[b]).wait()
    plsc.subcore_barrier()

    for i in range(RPT // K):
        b = i % NBUF
        if i >= NBUF:
            pltpu.make_async_copy(
                rows.at[b], out_hbm.at[cid, pl.ds(base, K)], sems.at[b]
            ).wait()
        pltpu.async_copy(acc.at[pl.ds(base + i * K, K)], rows.at[b], semg.at[b])
        pltpu.make_async_copy(acc.at[pl.ds(base, K)], rows.at[b], semg.at[b]).wait()
        pltpu.async_copy(rows.at[b], out_hbm.at[cid, pl.ds(base + i * K, K)], sems.at[b])
    for i in range(max(0, RPT // K - NBUF), RPT // K):
        b = i % NBUF
        pltpu.make_async_copy(
            rows.at[b], out_hbm.at[cid, pl.ds(base, K)], sems.at[b]
        ).wait()


RB = 2000
GB = N // RB


def _dinv_block(degt_blk):
    dsum = degt_blk[:, 0:1] + degt_blk[:, 1:2] + 1.0
    return jnp.broadcast_to(lax.rsqrt(dsum), (RB, D))


def _tc1_body(degt_ref, x_ref, w0_ref, b0_ref, w1_ref, us_ref):
    dinv_b = _dinv_block(degt_ref[...])
    h0 = jnp.dot(x_ref[...], w0_ref[...], preferred_element_type=jnp.float32)
    h0 = h0 + b0_ref[...]
    t = jnp.dot(h0, w1_ref[...], preferred_element_type=jnp.float32)
    u = t * dinv_b
    us_ref[0] = u[:, :DH]
    us_ref[1] = u[:, DH:]


_tc1 = pl.pallas_call(
    _tc1_body,
    grid=(GB,),
    in_specs=[
        pl.BlockSpec((RB, 2), lambda i: (i, 0)),
        pl.BlockSpec((RB, IND), lambda i: (i, 0)),
        pl.BlockSpec((IND, H0), lambda i: (0, 0)),
        pl.BlockSpec((1, H0), lambda i: (0, 0)),
        pl.BlockSpec((H0, D), lambda i: (0, 0)),
    ],
    out_specs=pl.BlockSpec((NC, RB, DH), lambda i: (0, i, 0)),
    out_shape=jax.ShapeDtypeStruct((NC, N, DH), jnp.float32),
)


def _agg_body(parts_ref, u_ref, degt_ref, b_ref, agg_ref, stats_ref):
    dinv_b = _dinv_block(degt_ref[...])
    scat = jnp.concatenate(
        [parts_ref[0] + u_ref[0], parts_ref[1] + u_ref[1]], axis=1
    )
    ag = dinv_b * scat + b_ref[...]
    agg_ref[...] = ag

    @pl.when(pl.program_id(0) == 0)
    def _():
        stats_ref[...] = jnp.zeros((2, D), jnp.float32)

    stats_ref[0:1, :] += jnp.sum(ag, axis=0, keepdims=True)
    stats_ref[1:2, :] += jnp.sum(ag * ag, axis=0, keepdims=True)


_tc_agg = pl.pallas_call(
    _agg_body,
    grid=(GB,),
    in_specs=[
        pl.BlockSpec((NC, RB, DH), lambda i: (0, i, 0)),
        pl.BlockSpec((NC, RB, DH), lambda i: (0, i, 0)),
        pl.BlockSpec((RB, 2), lambda i: (i, 0)),
        pl.BlockSpec((1, D), lambda i: (0, 0)),
    ],
    out_specs=[
        pl.BlockSpec((RB, D), lambda i: (i, 0)),
        pl.BlockSpec((2, D), lambda i: (0, 0)),
    ],
    out_shape=[
        jax.ShapeDtypeStruct((N, D), jnp.float32),
        jax.ShapeDtypeStruct((2, D), jnp.float32),
    ],
)


def _bn_mm_body(agg_ref, stats_ref, degt_ref, g_ref, be_ref, w_ref, us_ref):
    dinv_b = _dinv_block(degt_ref[...])
    m = stats_ref[0:1, :] * (1.0 / N)
    var = stats_ref[1:2, :] * (1.0 / N) - m * m
    sc = g_ref[...] * lax.rsqrt(var + EPS)
    h = (agg_ref[...] - m) * sc + be_ref[...]
    h = jnp.maximum(h, 0.0)
    t = jnp.dot(h, w_ref[...], preferred_element_type=jnp.float32)
    u = t * dinv_b
    us_ref[0] = u[:, :DH]
    us_ref[1] = u[:, DH:]


_tc_bn_mm = pl.pallas_call(
    _bn_mm_body,
    grid=(GB,),
    in_specs=[
        pl.BlockSpec((RB, D), lambda i: (i, 0)),
        pl.BlockSpec((2, D), lambda i: (0, 0)),
        pl.BlockSpec((RB, 2), lambda i: (i, 0)),
        pl.BlockSpec((1, D), lambda i: (0, 0)),
        pl.BlockSpec((1, D), lambda i: (0, 0)),
        pl.BlockSpec((D, D), lambda i: (0, 0)),
    ],
    out_specs=pl.BlockSpec((NC, RB, DH), lambda i: (0, i, 0)),
    out_shape=jax.ShapeDtypeStruct((NC, N, DH), jnp.float32),
)


def _head_body(agg_ref, stats_ref, g_ref, be_ref, w_ref, bo_ref, out_ref):
    m = stats_ref[0:1, :] * (1.0 / N)
    var = stats_ref[1:2, :] * (1.0 / N) - m * m
    sc = g_ref[...] * lax.rsqrt(var + EPS)
    h = (agg_ref[...] - m) * sc + be_ref[...]
    h = jnp.maximum(h, 0.0)
    z = jnp.dot(h, w_ref[...], preferred_element_type=jnp.float32) + bo_ref[...]
    zm = jnp.max(z, axis=1, keepdims=True)
    lse = jnp.log(jnp.sum(jnp.exp(z - zm), axis=1, keepdims=True))
    out_ref[...] = z - zm - lse


_tc_head = pl.pallas_call(
    _head_body,
    grid=(GB,),
    in_specs=[
        pl.BlockSpec((RB, D), lambda i: (i, 0)),
        pl.BlockSpec((2, D), lambda i: (0, 0)),
        pl.BlockSpec((1, D), lambda i: (0, 0)),
        pl.BlockSpec((1, D), lambda i: (0, 0)),
        pl.BlockSpec((D, NCLS), lambda i: (0, 0)),
        pl.BlockSpec((1, NCLS), lambda i: (0, 0)),
    ],
    out_specs=pl.BlockSpec((RB, NCLS), lambda i: (i, 0)),
    out_shape=jax.ShapeDtypeStruct((N, NCLS), jnp.float32),
)


def _make_conv(staged):
    return pl.kernel(
        functools.partial(_conv_body, staged),
        out_type=jax.ShapeDtypeStruct((NC, NPAD, DH), jnp.float32),
        mesh=_mesh,
        scratch_types=[
            pltpu.VMEM((CPT, K), jnp.int32),
            pltpu.VMEM((CPT, K), jnp.int32),
            pltpu.VMEM((NBUF, K, DH), jnp.float32),
            pltpu.VMEM((16, DH), jnp.float32),
            pltpu.VMEM_SHARED((NPAD, DH), jnp.float32),
            pltpu.VMEM_SHARED((N if staged else 8, DH), jnp.float32),
            pltpu.SemaphoreType.DMA((NBUF,)),
            pltpu.SemaphoreType.DMA((NBUF,)),
            pltpu.SemaphoreType.DMA,
        ],
        compiler_params=pltpu.CompilerParams(use_tc_tiling_on_sc=False),
    )


_conv_staged = _make_conv(True)
_conv_hbm = _make_conv(False)


def kernel(x, edge_index, W0, b0, W1, b1, W2, b2, g1, be1, g2, be2, Wout, bout):
    src = edge_index[0].astype(jnp.int32)
    dst = edge_index[1].astype(jnp.int32)
    pad = EPAD - E
    src2d = jnp.concatenate([src, jnp.zeros((pad,), jnp.int32)]).reshape(ROWS2D, K)
    dst2d = jnp.concatenate(
        [dst, N + (jnp.arange(pad, dtype=jnp.int32) % NPADROWS)]
    ).reshape(ROWS2D, K)
    srcb = jnp.concatenate([src2d, src2d + N], axis=0)

    degp = _deg_kernel(dst2d).reshape(NC, NPAD)
    degt = degp.T

    u1s = _tc1(degt, x, W0, b0.reshape(1, H0), W1)
    parts1 = _conv_hbm(u1s.reshape(NC * N, DH), srcb, dst2d)
    agg1, stats1 = _tc_agg(parts1, u1s, degt, b1.reshape(1, D))
    u2s = _tc_bn_mm(
        agg1, stats1, degt, g1.reshape(1, D), be1.reshape(1, D), W2
    )
    parts2 = _conv_hbm(u2s.reshape(NC * N, DH), srcb, dst2d)
    agg2, stats2 = _tc_agg(parts2, u2s, degt, b2.reshape(1, D))
    return _tc_head(
        agg2, stats2, g2.reshape(1, D), be2.reshape(1, D), Wout, bout.reshape(1, NCLS)
    )


# speedup vs baseline: 1.0602x; 1.0602x over previous
import functools

import jax
import jax.numpy as jnp
from jax import lax
from jax.experimental import pallas as pl
from jax.experimental.pallas import tpu as pltpu
from jax.experimental.pallas import tpu_sc as plsc

N = 10000
E = 320000
IND = 128
H0 = 512
D = 128
NCLS = 16
EPS = 1e-5

NC = 2
NS = 16
NW = NC * NS
K = 128
RPW = 80
ROWS2D = NW * RPW
EPAD = ROWS2D * K
NPADROWS = 240
NPAD = N + NPADROWS
RPT = NPAD // NS
NBUF = 5
LOOKAHEAD = 3
CPT = ROWS2D // NS
DH = D // 2

_mesh = plsc.VectorSubcoreMesh(
    core_axis_name="c", subcore_axis_name="s", num_cores=NC, num_subcores=NS
)


@functools.partial(
    pl.kernel,
    out_type=jax.ShapeDtypeStruct((NC * NPAD,), jnp.float32),
    mesh=_mesh,
    scratch_types=[
        pltpu.VMEM((RPW, K), jnp.int32),
        pltpu.VMEM((K,), jnp.float32),
        pltpu.VMEM((640,), jnp.float32),
        pltpu.VMEM_SHARED((NPAD,), jnp.float32),
    ],
)
def _deg_kernel(dst_hbm, out_hbm, dstv, ones_v, buf_v, acc):
    cid = lax.axis_index("c")
    sid = lax.axis_index("s")
    wid = sid * NC + cid
    for i in range(K // 16):
        ones_v[pl.ds(i * 16, 16)] = jnp.ones((16,), jnp.float32)
    for i in range(640 // 16):
        buf_v[pl.ds(i * 16, 16)] = jnp.zeros((16,), jnp.float32)

    pltpu.sync_copy(buf_v.at[pl.ds(0, RPT)], acc.at[pl.ds(sid * RPT, RPT)])

    pltpu.sync_copy(dst_hbm.at[pl.ds(wid * RPW, RPW)], dstv)
    plsc.subcore_barrier()

    def body(j, carry):
        pltpu.sync_copy(ones_v, acc.at[dstv.at[j]], add=True)
        return carry

    lax.fori_loop(0, RPW, body, 0)
    plsc.subcore_barrier()

    pltpu.sync_copy(acc.at[pl.ds(sid * RPT, RPT)], buf_v.at[pl.ds(0, RPT)])
    pltpu.sync_copy(
        buf_v.at[pl.ds(0, RPT)], out_hbm.at[pl.ds(cid * NPAD + sid * RPT, RPT)]
    )


def _conv_body(staged, u_hbm, src_hbm, dst_hbm, out_hbm, srcv, dstv, rows, zbuf,
               acc, stage, semg, sems, semz):
    cid = lax.axis_index("c")
    sid = lax.axis_index("s")
    base = sid * RPT

    if staged:
        pltpu.sync_copy(src_hbm.at[pl.ds(sid * CPT, CPT)], srcv)
    else:
        pltpu.sync_copy(src_hbm.at[pl.ds(cid * ROWS2D + sid * CPT, CPT)], srcv)
    pltpu.sync_copy(dst_hbm.at[pl.ds(sid * CPT, CPT)], dstv)
    if staged:
        sr0 = sid * 632

        @pl.when(sid < 15)
        def _():
            pltpu.sync_copy(
                u_hbm.at[pl.ds(cid * N + sr0, 632)], stage.at[pl.ds(sr0, 632)]
            )

        @pl.when(sid == 15)
        def _():
            pltpu.sync_copy(
                u_hbm.at[pl.ds(cid * N + 15 * 632, 520)],
                stage.at[pl.ds(15 * 632, 520)],
            )
    else:
        stage = u_hbm
        for b in range(LOOKAHEAD):
            pltpu.async_copy(stage.at[srcv.at[b]], rows.at[b], semg.at[b])

    for r in range(16):
        for c in range(DH // 16):
            zbuf[r, pl.ds(c * 16, 16)] = jnp.zeros((16,), jnp.float32)

    def zero_start(i, carry):
        pltpu.async_copy(zbuf, acc.at[pl.ds(base + i * 16, 16)], semz)
        return carry

    lax.fori_loop(0, RPT // 16, zero_start, 0)

    def zero_drain(i, carry):
        pltpu.make_async_copy(zbuf, acc.at[pl.ds(base, 16)], semz).wait()
        return carry

    lax.fori_loop(0, RPT // 16, zero_drain, 0)
    plsc.subcore_barrier()
    if staged:
        for b in range(LOOKAHEAD):
            pltpu.async_copy(stage.at[srcv.at[b]], rows.at[b], semg.at[b])

    def step(g, carry):
        for b in range(NBUF):
            j = g * NBUF + b
            pltpu.make_async_copy(stage.at[srcv.at[j]], rows.at[b], semg.at[b]).wait()
            pltpu.async_copy(rows.at[b], acc.at[dstv.at[j]], sems.at[b], add=True)
            jj = j + LOOKAHEAD
            bb = (b + LOOKAHEAD) % NBUF

            @pl.when(jj < CPT)
            def _():
                @pl.when(j >= NBUF - LOOKAHEAD)
                def _():
                    pltpu.make_async_copy(
                        rows.at[bb], acc.at[dstv.at[j]], sems.at[bb]
                    ).wait()

                pltpu.async_copy(stage.at[srcv.at[jj]], rows.at[bb], semg.at[bb])

        return carry

    lax.fori_loop(0, CPT // NBUF, step, 0)
    for b in range(NBUF):
        pltpu.make_async_copy(rows.at[b], acc.at[dstv.at[0]], sems.at[b]).wait()
    plsc.subcore_barrier()

    for i in range(RPT // K):
        b = i % NBUF
        if i >= NBUF:
            pltpu.make_async_copy(
                rows.at[b], out_hbm.at[cid, pl.ds(base, K)], sems.at[b]
            ).wait()
        pltpu.async_copy(acc.at[pl.ds(base + i * K, K)], rows.at[b], semg.at[b])
        pltpu.make_async_copy(acc.at[pl.ds(base, K)], rows.at[b], semg.at[b]).wait()
        pltpu.async_copy(rows.at[b], out_hbm.at[cid, pl.ds(base + i * K, K)], sems.at[b])
    for i in range(max(0, RPT // K - NBUF), RPT // K):
        b = i % NBUF
        pltpu.make_async_copy(
            rows.at[b], out_hbm.at[cid, pl.ds(base, K)], sems.at[b]
        ).wait()


RB = 2000
GB = N // RB


def _dinv_block(degt_blk):
    dsum = degt_blk[:, 0:1] + degt_blk[:, 1:2] + 1.0
    return jnp.broadcast_to(lax.rsqrt(dsum), (RB, D))


def _tc1_body(degt_ref, x_ref, w0_ref, b0_ref, w1_ref, u1_ref, us_ref):
    dinv_b = _dinv_block(degt_ref[...])
    h0 = jnp.dot(x_ref[...], w0_ref[...], preferred_element_type=jnp.float32)
    h0 = h0 + b0_ref[...]
    t = jnp.dot(h0, w1_ref[...], preferred_element_type=jnp.float32)
    u = t * dinv_b
    u1_ref[...] = u
    us_ref[0] = u[:, :DH]
    us_ref[1] = u[:, DH:]


_tc1 = pl.pallas_call(
    _tc1_body,
    grid=(GB,),
    in_specs=[
        pl.BlockSpec((RB, 2), lambda i: (i, 0)),
        pl.BlockSpec((RB, IND), lambda i: (i, 0)),
        pl.BlockSpec((IND, H0), lambda i: (0, 0)),
        pl.BlockSpec((1, H0), lambda i: (0, 0)),
        pl.BlockSpec((H0, D), lambda i: (0, 0)),
    ],
    out_specs=[
        pl.BlockSpec((RB, D), lambda i: (i, 0)),
        pl.BlockSpec((NC, RB, DH), lambda i: (0, i, 0)),
    ],
    out_shape=[
        jax.ShapeDtypeStruct((N, D), jnp.float32),
        jax.ShapeDtypeStruct((NC, N, DH), jnp.float32),
    ],
)


def _agg_body(parts_ref, u_ref, degt_ref, b_ref, agg_ref, stats_ref):
    dinv_b = _dinv_block(degt_ref[...])
    scat = jnp.concatenate([parts_ref[0], parts_ref[1]], axis=1)
    ag = dinv_b * (scat + u_ref[...]) + b_ref[...]
    agg_ref[...] = ag

    @pl.when(pl.program_id(0) == 0)
    def _():
        stats_ref[...] = jnp.zeros((2, D), jnp.float32)

    stats_ref[0:1, :] += jnp.sum(ag, axis=0, keepdims=True)
    stats_ref[1:2, :] += jnp.sum(ag * ag, axis=0, keepdims=True)


_tc_agg = pl.pallas_call(
    _agg_body,
    grid=(GB,),
    in_specs=[
        pl.BlockSpec((NC, RB, DH), lambda i: (0, i, 0)),
        pl.BlockSpec((RB, D), lambda i: (i, 0)),
        pl.BlockSpec((RB, 2), lambda i: (i, 0)),
        pl.BlockSpec((1, D), lambda i: (0, 0)),
    ],
    out_specs=[
        pl.BlockSpec((RB, D), lambda i: (i, 0)),
        pl.BlockSpec((2, D), lambda i: (0, 0)),
    ],
    out_shape=[
        jax.ShapeDtypeStruct((N, D), jnp.float32),
        jax.ShapeDtypeStruct((2, D), jnp.float32),
    ],
)


def _bn_mm_body(agg_ref, stats_ref, degt_ref, g_ref, be_ref, w_ref, u_ref, us_ref):
    dinv_b = _dinv_block(degt_ref[...])
    m = stats_ref[0:1, :] * (1.0 / N)
    var = stats_ref[1:2, :] * (1.0 / N) - m * m
    sc = g_ref[...] * lax.rsqrt(var + EPS)
    h = (agg_ref[...] - m) * sc + be_ref[...]
    h = jnp.maximum(h, 0.0)
    t = jnp.dot(h, w_ref[...], preferred_element_type=jnp.float32)
    u = t * dinv_b
    u_ref[...] = u
    us_ref[0] = u[:, :DH]
    us_ref[1] = u[:, DH:]


_tc_bn_mm = pl.pallas_call(
    _bn_mm_body,
    grid=(GB,),
    in_specs=[
        pl.BlockSpec((RB, D), lambda i: (i, 0)),
        pl.BlockSpec((2, D), lambda i: (0, 0)),
        pl.BlockSpec((RB, 2), lambda i: (i, 0)),
        pl.BlockSpec((1, D), lambda i: (0, 0)),
        pl.BlockSpec((1, D), lambda i: (0, 0)),
        pl.BlockSpec((D, D), lambda i: (0, 0)),
    ],
    out_specs=[
        pl.BlockSpec((RB, D), lambda i: (i, 0)),
        pl.BlockSpec((NC, RB, DH), lambda i: (0, i, 0)),
    ],
    out_shape=[
        jax.ShapeDtypeStruct((N, D), jnp.float32),
        jax.ShapeDtypeStruct((NC, N, DH), jnp.float32),
    ],
)


def _head_body(agg_ref, stats_ref, g_ref, be_ref, w_ref, bo_ref, out_ref):
    m = stats_ref[0:1, :] * (1.0 / N)
    var = stats_ref[1:2, :] * (1.0 / N) - m * m
    sc = g_ref[...] * lax.rsqrt(var + EPS)
    h = (agg_ref[...] - m) * sc + be_ref[...]
    h = jnp.maximum(h, 0.0)
    z = jnp.dot(h, w_ref[...], preferred_element_type=jnp.float32) + bo_ref[...]
    zm = jnp.max(z, axis=1, keepdims=True)
    lse = jnp.log(jnp.sum(jnp.exp(z - zm), axis=1, keepdims=True))
    out_ref[...] = z - zm - lse


_tc_head = pl.pallas_call(
    _head_body,
    grid=(GB,),
    in_specs=[
        pl.BlockSpec((RB, D), lambda i: (i, 0)),
        pl.BlockSpec((2, D), lambda i: (0, 0)),
        pl.BlockSpec((1, D), lambda i: (0, 0)),
        pl.BlockSpec((1, D), lambda i: (0, 0)),
        pl.BlockSpec((D, NCLS), lambda i: (0, 0)),
        pl.BlockSpec((1, NCLS), lambda i: (0, 0)),
    ],
    out_specs=pl.BlockSpec((RB, NCLS), lambda i: (i, 0)),
    out_shape=jax.ShapeDtypeStruct((N, NCLS), jnp.float32),
)


def _make_conv(staged):
    return pl.kernel(
        functools.partial(_conv_body, staged),
        out_type=jax.ShapeDtypeStruct((NC, NPAD, DH), jnp.float32),
        mesh=_mesh,
        scratch_types=[
            pltpu.VMEM((CPT, K), jnp.int32),
            pltpu.VMEM((CPT, K), jnp.int32),
            pltpu.VMEM((NBUF, K, DH), jnp.float32),
            pltpu.VMEM((16, DH), jnp.float32),
            pltpu.VMEM_SHARED((NPAD, DH), jnp.float32),
            pltpu.VMEM_SHARED((N if staged else 8, DH), jnp.float32),
            pltpu.SemaphoreType.DMA((NBUF,)),
            pltpu.SemaphoreType.DMA((NBUF,)),
            pltpu.SemaphoreType.DMA,
        ],
        compiler_params=pltpu.CompilerParams(use_tc_tiling_on_sc=False),
    )


_conv_staged = _make_conv(True)
_conv_hbm = _make_conv(False)


def kernel(x, edge_index, W0, b0, W1, b1, W2, b2, g1, be1, g2, be2, Wout, bout):
    src = edge_index[0].astype(jnp.int32)
    dst = edge_index[1].astype(jnp.int32)
    pad = EPAD - E
    src2d = jnp.concatenate([src, jnp.zeros((pad,), jnp.int32)]).reshape(ROWS2D, K)
    dst2d = jnp.concatenate(
        [dst, N + (jnp.arange(pad, dtype=jnp.int32) % NPADROWS)]
    ).reshape(ROWS2D, K)
    srcb = jnp.concatenate([src2d, src2d + N], axis=0)

    degp = _deg_kernel(dst2d).reshape(NC, NPAD)
    degt = degp.T

    u1, u1s = _tc1(degt, x, W0, b0.reshape(1, H0), W1)
    parts1 = _conv_hbm(u1s.reshape(NC * N, DH), srcb, dst2d)
    agg1, stats1 = _tc_agg(parts1, u1, degt, b1.reshape(1, D))
    u2, u2s = _tc_bn_mm(
        agg1, stats1, degt, g1.reshape(1, D), be1.reshape(1, D), W2
    )
    parts2 = _conv_hbm(u2s.reshape(NC * N, DH), srcb, dst2d)
    agg2, stats2 = _tc_agg(parts2, u2, degt, b2.reshape(1, D))
    return _tc_head(
        agg2, stats2, g2.reshape(1, D), be2.reshape(1, D), Wout, bout.reshape(1, NCLS)
    )
